# baseline (device time: 83230 ns/iter reference)
import math

import jax
import jax.numpy as jnp
from jax import lax
from jax.experimental import pallas as pl
from jax.experimental.pallas import tpu as pltpu

N_DEV = 16
B_LOC = 2
SQ = 128
D = 512
H_LOC = 4
DH = 64
B_GLB = N_DEV * B_LOC
ROWS = B_GLB * SQ
DH_LOC = H_LOC * DH

RING = [0, 4, 8, 12, 15, 11, 7, 3, 2, 6, 10, 14, 13, 9, 5, 1]
RING_INV = [RING.index(p) for p in range(N_DEV)]


def kernel(x, Wq, Wk, Wv, Wo):
    def body(x_ref, wq_ref, wk_ref, wv_ref, wo_ref, out_ref,
             xg, part, rsb_r, rsb_l,
             agr_send, agr_recv, agl_send, agl_recv,
             rsr_send, rsr_recv, rsl_send, rsl_recv):
        my = lax.axis_index("i")

        def m16(v):
            return lax.rem(v + 2 * N_DEV, N_DEV)

        def lut(table, idx):
            acc = jnp.int32(0)
            for j, v in enumerate(table):
                acc = acc + jnp.where(idx == j, jnp.int32(v), jnp.int32(0))
            return acc

        my_r = lut(RING_INV, my)
        left = lut(RING, m16(my_r - 1))
        right = lut(RING, m16(my_r + 1))

        def erow(c):
            return c * (2 * SQ)

        def orow(c):
            return c * (2 * SQ) + SQ

        barrier = pltpu.get_barrier_semaphore()
        for nbr in (left, right):
            pl.semaphore_signal(barrier, inc=1, device_id=(nbr,),
                                device_id_type=pl.DeviceIdType.MESH)
        pl.semaphore_wait(barrier, 2)

        xg[pl.ds(erow(my), SQ), :] = x_ref[0, :, :].astype(jnp.bfloat16)
        xg[pl.ds(orow(my), SQ), :] = x_ref[1, :, :].astype(jnp.bfloat16)

        rows_i = lax.broadcasted_iota(jnp.int32, (SQ, DH_LOC), 0)
        cols_i = lax.broadcasted_iota(jnp.int32, (SQ, DH_LOC), 1)
        k2 = ((jnp.remainder(cols_i, DH) // 2) * 2).astype(jnp.float32)
        inv = jnp.exp(k2 * (-math.log(10000.0) / DH))
        ang = rows_i.astype(jnp.float32) * inv
        cosv = jnp.cos(ang)
        sinv = jnp.sin(ang)

        ri = lax.broadcasted_iota(jnp.int32, (DH_LOC, DH_LOC), 0)
        ci = lax.broadcasted_iota(jnp.int32, (DH_LOC, DH_LOC), 1)
        even_c = jnp.remainder(ci, 2) == 0
        rot_m = (jnp.where(even_c & (ri == ci + 1), -1.0, 0.0)
                 + jnp.where((~even_c) & (ri == ci - 1), 1.0, 0.0)
                 ).astype(jnp.float32)

        cos2 = jnp.concatenate([cosv, cosv], axis=0)
        sin2 = jnp.concatenate([sinv, sinv], axis=0)

        wq = wq_ref[...].astype(jnp.bfloat16)
        wk = wk_ref[...].astype(jnp.bfloat16)
        wv = wv_ref[...].astype(jnp.bfloat16)
        wo = wo_ref[...].astype(jnp.bfloat16)

        def compute_partial(row_e, row_o):
            xb = jnp.concatenate(
                [xg[pl.ds(row_e, SQ), :], xg[pl.ds(row_o, SQ), :]], axis=0)
            q = jnp.dot(xb, wq, preferred_element_type=jnp.float32)
            k = jnp.dot(xb, wk, preferred_element_type=jnp.float32)
            v = jnp.dot(xb, wv, preferred_element_type=jnp.float32)
            q = q * cos2 + jnp.dot(q, rot_m,
                                   preferred_element_type=jnp.float32) * sin2
            k = k * cos2 + jnp.dot(k, rot_m,
                                   preferred_element_type=jnp.float32) * sin2
            q = q.astype(jnp.bfloat16)
            k = k.astype(jnp.bfloat16)
            v = v.astype(jnp.bfloat16)
            halves = []
            for bb in range(2):
                ctx_heads = []
                for hh in range(H_LOC):
                    qs = q[bb * SQ:(bb + 1) * SQ, hh * DH:(hh + 1) * DH]
                    ks = k[bb * SQ:(bb + 1) * SQ, hh * DH:(hh + 1) * DH]
                    vs = v[bb * SQ:(bb + 1) * SQ, hh * DH:(hh + 1) * DH]
                    s = lax.dot_general(
                        qs, ks, (((1,), (1,)), ((), ())),
                        preferred_element_type=jnp.float32) * 0.125
                    mx = jnp.max(s, axis=-1, keepdims=True)
                    w = jnp.exp(s - mx)
                    w = (w / jnp.sum(w, axis=-1, keepdims=True)
                         ).astype(jnp.bfloat16)
                    ctx_heads.append(jnp.dot(
                        w, vs,
                        preferred_element_type=jnp.float32
                    ).astype(jnp.bfloat16))
                halves.append(jnp.concatenate(ctx_heads, axis=1))
            ctx = jnp.concatenate(halves, axis=0)
            return jnp.dot(ctx, wo, preferred_element_type=jnp.float32)

        def remote(src, dst, ssem, rsem, dev):
            return pltpu.make_async_remote_copy(
                src_ref=src, dst_ref=dst, send_sem=ssem, recv_sem=rsem,
                device_id=(dev,), device_id_type=pl.DeviceIdType.MESH)

        HD = D // 2

        agr_d = agl_d = None
        for h in range(N_DEV + 1):
            if h < N_DEV:
                re_h = erow(lut(RING, m16(my_r - h)))
                ro_h = orow(lut(RING, m16(my_r + h)))

            if h < N_DEV - 1:
                if agr_d is not None:
                    agr_d.wait_recv()
                new_agr = remote(xg.at[pl.ds(re_h, SQ)],
                                 xg.at[pl.ds(re_h, SQ)],
                                 agr_send.at[h], agr_recv.at[h], right)
                new_agr.start()
                if agl_d is not None:
                    agl_d.wait_recv()
                new_agl = remote(xg.at[pl.ds(ro_h, SQ)],
                                 xg.at[pl.ds(ro_h, SQ)],
                                 agl_send.at[h], agl_recv.at[h], left)
                new_agl.start()
            elif h == N_DEV - 1:
                agr_d.wait_recv()
                agl_d.wait_recv()
                new_agr = new_agl = None

            if h >= 2:
                t = h - 2
                src_r = (part.at[pl.ds(erow(lut(RING, m16(my_r - 1))), SQ)]
                         if t == 0 else rsb_r.at[t - 1])
                new_rsr = remote(src_r, rsb_r.at[t],
                                 rsr_send.at[t], rsr_recv.at[t], right)
                new_rsr.start()
                src_l = (part.at[pl.ds(orow(lut(RING, m16(my_r + 1))), SQ)]
                         if t == 0 else rsb_l.at[t - 1])
                new_rsl = remote(src_l, rsb_l.at[t],
                                 rsl_send.at[t], rsl_recv.at[t], left)
                new_rsl.start()

            if h < N_DEV:
                po = compute_partial(re_h, ro_h)
                if h <= 1:
                    part[pl.ds(re_h, SQ), :] = po[0:SQ, :].astype(jnp.bfloat16)
                    part[pl.ds(ro_h, SQ), :] = po[SQ:2 * SQ, :].astype(
                        jnp.bfloat16)

            if h >= 2:
                t = h - 2
                if h < N_DEV:
                    add_e = po[0:SQ, :]
                    add_o = po[SQ:2 * SQ, :]
                else:
                    add_e = part[pl.ds(erow(my), SQ), :].astype(jnp.float32)
                    add_o = part[pl.ds(orow(my), SQ), :].astype(jnp.float32)
                new_rsr.wait_recv()
                rsb_r[t] = (rsb_r[t].astype(jnp.float32)
                            + add_e).astype(jnp.bfloat16)
                new_rsl.wait_recv()
                rsb_l[t] = (rsb_l[t].astype(jnp.float32)
                            + add_o).astype(jnp.bfloat16)

            if h < N_DEV - 1:
                new_agr.wait_send()
                new_agl.wait_send()
                agr_d, agl_d = new_agr, new_agl
            if h >= 2:
                new_rsr.wait_send()
                new_rsl.wait_send()

        out_ref[0, :, :] = rsb_r[N_DEV - 2].astype(jnp.float32)
        out_ref[1, :, :] = rsb_l[N_DEV - 2].astype(jnp.float32)

    return pl.pallas_call(
        body,
        out_shape=jax.ShapeDtypeStruct((B_LOC, SQ, D), jnp.float32),
        in_specs=[pl.BlockSpec(memory_space=pltpu.VMEM)] * 5,
        out_specs=pl.BlockSpec(memory_space=pltpu.VMEM),
        scratch_shapes=[
            pltpu.VMEM((ROWS, D), jnp.bfloat16),
            pltpu.VMEM((ROWS, D), jnp.bfloat16),
            pltpu.VMEM((N_DEV - 1, SQ, D), jnp.bfloat16),
            pltpu.VMEM((N_DEV - 1, SQ, D), jnp.bfloat16),
            pltpu.SemaphoreType.DMA((N_DEV - 1,)),
            pltpu.SemaphoreType.DMA((N_DEV - 1,)),
            pltpu.SemaphoreType.DMA((N_DEV - 1,)),
            pltpu.SemaphoreType.DMA((N_DEV - 1,)),
            pltpu.SemaphoreType.DMA((N_DEV - 1,)),
            pltpu.SemaphoreType.DMA((N_DEV - 1,)),
            pltpu.SemaphoreType.DMA((N_DEV - 1,)),
            pltpu.SemaphoreType.DMA((N_DEV - 1,)),
        ],
        compiler_params=pltpu.CompilerParams(
            collective_id=0,
            vmem_limit_bytes=63 * 1024 * 1024,
        ),
    )(x, Wq, Wk, Wv, Wo)


# device time: 75702 ns/iter; 1.0994x vs baseline; 1.0994x over previous
import math

import jax
import jax.numpy as jnp
from jax import lax
from jax.experimental import pallas as pl
from jax.experimental.pallas import tpu as pltpu

N_DEV = 16
B_LOC = 2
SQ = 128
D = 512
H_LOC = 4
DH = 64
B_GLB = N_DEV * B_LOC
ROWS = B_GLB * SQ
DH_LOC = H_LOC * DH

RING = [0, 4, 8, 12, 15, 11, 7, 3, 2, 6, 10, 14, 13, 9, 5, 1]
RING_INV = [RING.index(p) for p in range(N_DEV)]


def kernel(x, Wq, Wk, Wv, Wo):
    def body(x_ref, wq_ref, wk_ref, wv_ref, wo_ref, out_ref,
             xg, part, rsb_r, rsb_l,
             agr_send, agr_recv, agl_send, agl_recv,
             rsr_send, rsr_recv, rsl_send, rsl_recv):
        my = lax.axis_index("i")

        def m16(v):
            return lax.rem(v + 2 * N_DEV, N_DEV)

        def lut(table, idx):
            acc = jnp.int32(0)
            for j, v in enumerate(table):
                acc = acc + jnp.where(idx == j, jnp.int32(v), jnp.int32(0))
            return acc

        my_r = lut(RING_INV, my)
        left = lut(RING, m16(my_r - 1))
        right = lut(RING, m16(my_r + 1))

        def erow(c):
            return c * (2 * SQ)

        def orow(c):
            return c * (2 * SQ) + SQ

        barrier = pltpu.get_barrier_semaphore()
        for nbr in (left, right):
            pl.semaphore_signal(barrier, inc=1, device_id=(nbr,),
                                device_id_type=pl.DeviceIdType.MESH)
        pl.semaphore_wait(barrier, 2)

        xg[pl.ds(erow(my), SQ), :] = x_ref[0, :, :].astype(jnp.bfloat16)
        xg[pl.ds(orow(my), SQ), :] = x_ref[1, :, :].astype(jnp.bfloat16)

        rows_i = lax.broadcasted_iota(jnp.int32, (SQ, DH_LOC), 0)
        cols_i = lax.broadcasted_iota(jnp.int32, (SQ, DH_LOC), 1)
        k2 = ((jnp.remainder(cols_i, DH) // 2) * 2).astype(jnp.float32)
        inv = jnp.exp(k2 * (-math.log(10000.0) / DH))
        ang = rows_i.astype(jnp.float32) * inv
        cosv = jnp.cos(ang)
        sinv = jnp.sin(ang)

        ri = lax.broadcasted_iota(jnp.int32, (DH_LOC, DH_LOC), 0)
        ci = lax.broadcasted_iota(jnp.int32, (DH_LOC, DH_LOC), 1)
        even_c = jnp.remainder(ci, 2) == 0
        rot_m = (jnp.where(even_c & (ri == ci + 1), -1.0, 0.0)
                 + jnp.where((~even_c) & (ri == ci - 1), 1.0, 0.0)
                 ).astype(jnp.float32)

        cos2 = jnp.concatenate([cosv, cosv], axis=0)
        sin2 = jnp.concatenate([sinv, sinv], axis=0)

        wq = wq_ref[...].astype(jnp.bfloat16)
        wk = wk_ref[...].astype(jnp.bfloat16)
        wv = wv_ref[...].astype(jnp.bfloat16)
        wo = wo_ref[...].astype(jnp.bfloat16)

        def compute_partial(row_e, row_o):
            xb = jnp.concatenate(
                [xg[pl.ds(row_e, SQ), :], xg[pl.ds(row_o, SQ), :]], axis=0)
            q = jnp.dot(xb, wq, preferred_element_type=jnp.float32)
            k = jnp.dot(xb, wk, preferred_element_type=jnp.float32)
            v = jnp.dot(xb, wv, preferred_element_type=jnp.float32)
            q = q * cos2 + jnp.dot(q, rot_m,
                                   preferred_element_type=jnp.float32) * sin2
            k = k * cos2 + jnp.dot(k, rot_m,
                                   preferred_element_type=jnp.float32) * sin2
            q = q.astype(jnp.bfloat16)
            k = k.astype(jnp.bfloat16)
            v = v.astype(jnp.bfloat16)
            halves = []
            for bb in range(2):
                ctx_heads = []
                for hh in range(H_LOC):
                    qs = q[bb * SQ:(bb + 1) * SQ, hh * DH:(hh + 1) * DH]
                    ks = k[bb * SQ:(bb + 1) * SQ, hh * DH:(hh + 1) * DH]
                    vs = v[bb * SQ:(bb + 1) * SQ, hh * DH:(hh + 1) * DH]
                    s = lax.dot_general(
                        qs, ks, (((1,), (1,)), ((), ())),
                        preferred_element_type=jnp.float32) * 0.125
                    mx = jnp.max(s, axis=-1, keepdims=True)
                    w = jnp.exp(s - mx)
                    w = (w / jnp.sum(w, axis=-1, keepdims=True)
                         ).astype(jnp.bfloat16)
                    ctx_heads.append(jnp.dot(
                        w, vs,
                        preferred_element_type=jnp.float32
                    ).astype(jnp.bfloat16))
                halves.append(jnp.concatenate(ctx_heads, axis=1))
            ctx = jnp.concatenate(halves, axis=0)
            return jnp.dot(ctx, wo, preferred_element_type=jnp.float32)

        def remote(src, dst, ssem, rsem, dev):
            return pltpu.make_async_remote_copy(
                src_ref=src, dst_ref=dst, send_sem=ssem, recv_sem=rsem,
                device_id=(dev,), device_id_type=pl.DeviceIdType.MESH)

        HD = D // 2

        agr_d = agl_d = rsr_d = rsl_d = None
        for h in range(N_DEV + 1):
            if h < N_DEV:
                re_h = erow(lut(RING, m16(my_r - h)))
                ro_h = orow(lut(RING, m16(my_r + h)))

            if h < N_DEV - 1:
                if agr_d is not None:
                    agr_d.wait_recv()
                new_agr = remote(xg.at[pl.ds(re_h, SQ)],
                                 xg.at[pl.ds(re_h, SQ)],
                                 agr_send.at[h], agr_recv.at[h], right)
                new_agr.start()
                if agl_d is not None:
                    agl_d.wait_recv()
                new_agl = remote(xg.at[pl.ds(ro_h, SQ)],
                                 xg.at[pl.ds(ro_h, SQ)],
                                 agl_send.at[h], agl_recv.at[h], left)
                new_agl.start()
            elif h == N_DEV - 1:
                agr_d.wait_recv()
                agl_d.wait_recv()
                new_agr = new_agl = None

            if h < N_DEV:
                po = compute_partial(re_h, ro_h)
                if h <= 1:
                    part[pl.ds(re_h, SQ), :] = po[0:SQ, :].astype(jnp.bfloat16)
                    part[pl.ds(ro_h, SQ), :] = po[SQ:2 * SQ, :].astype(
                        jnp.bfloat16)

            if h >= 2:
                t = h - 2
                if h < N_DEV:
                    add_e = po[0:SQ, :]
                    add_o = po[SQ:2 * SQ, :]
                else:
                    add_e = part[pl.ds(erow(my), SQ), :].astype(jnp.float32)
                    add_o = part[pl.ds(orow(my), SQ), :].astype(jnp.float32)
                rsr_d.wait_recv()
                rsb_r[t] = (rsb_r[t].astype(jnp.float32)
                            + add_e).astype(jnp.bfloat16)
                rsl_d.wait_recv()
                rsb_l[t] = (rsb_l[t].astype(jnp.float32)
                            + add_o).astype(jnp.bfloat16)
                rsr_d.wait_send()
                rsl_d.wait_send()

            if 1 <= h <= N_DEV - 1:
                t2 = h - 1
                src_r = (part.at[pl.ds(erow(lut(RING, m16(my_r - 1))), SQ)]
                         if t2 == 0 else rsb_r.at[t2 - 1])
                rsr_d = remote(src_r, rsb_r.at[t2],
                               rsr_send.at[t2], rsr_recv.at[t2], right)
                rsr_d.start()
                src_l = (part.at[pl.ds(orow(lut(RING, m16(my_r + 1))), SQ)]
                         if t2 == 0 else rsb_l.at[t2 - 1])
                rsl_d = remote(src_l, rsb_l.at[t2],
                               rsl_send.at[t2], rsl_recv.at[t2], left)
                rsl_d.start()

            if h < N_DEV - 1:
                new_agr.wait_send()
                new_agl.wait_send()
                agr_d, agl_d = new_agr, new_agl

        out_ref[0, :, :] = rsb_r[N_DEV - 2].astype(jnp.float32)
        out_ref[1, :, :] = rsb_l[N_DEV - 2].astype(jnp.float32)

    return pl.pallas_call(
        body,
        out_shape=jax.ShapeDtypeStruct((B_LOC, SQ, D), jnp.float32),
        in_specs=[pl.BlockSpec(memory_space=pltpu.VMEM)] * 5,
        out_specs=pl.BlockSpec(memory_space=pltpu.VMEM),
        scratch_shapes=[
            pltpu.VMEM((ROWS, D), jnp.bfloat16),
            pltpu.VMEM((ROWS, D), jnp.bfloat16),
            pltpu.VMEM((N_DEV - 1, SQ, D), jnp.bfloat16),
            pltpu.VMEM((N_DEV - 1, SQ, D), jnp.bfloat16),
            pltpu.SemaphoreType.DMA((N_DEV - 1,)),
            pltpu.SemaphoreType.DMA((N_DEV - 1,)),
            pltpu.SemaphoreType.DMA((N_DEV - 1,)),
            pltpu.SemaphoreType.DMA((N_DEV - 1,)),
            pltpu.SemaphoreType.DMA((N_DEV - 1,)),
            pltpu.SemaphoreType.DMA((N_DEV - 1,)),
            pltpu.SemaphoreType.DMA((N_DEV - 1,)),
            pltpu.SemaphoreType.DMA((N_DEV - 1,)),
        ],
        compiler_params=pltpu.CompilerParams(
            collective_id=0,
            vmem_limit_bytes=63 * 1024 * 1024,
        ),
    )(x, Wq, Wk, Wv, Wo)


# device time: 74483 ns/iter; 1.1174x vs baseline; 1.0164x over previous
import math

import jax
import jax.numpy as jnp
from jax import lax
from jax.experimental import pallas as pl
from jax.experimental.pallas import tpu as pltpu

N_DEV = 16
B_LOC = 2
SQ = 128
D = 512
H_LOC = 4
DH = 64
B_GLB = N_DEV * B_LOC
ROWS = B_GLB * SQ
DH_LOC = H_LOC * DH

RING = [0, 4, 8, 12, 15, 11, 7, 3, 2, 6, 10, 14, 13, 9, 5, 1]
RING_INV = [RING.index(p) for p in range(N_DEV)]


def kernel(x, Wq, Wk, Wv, Wo):
    def body(x_ref, wq_ref, wk_ref, wv_ref, wo_ref, out_ref,
             xg, part, rsb_r, rsb_l,
             agr_send, agr_recv, agl_send, agl_recv,
             rsr_send, rsr_recv, rsl_send, rsl_recv):
        my = lax.axis_index("i")

        def m16(v):
            return lax.rem(v + 2 * N_DEV, N_DEV)

        def lut(table, idx):
            acc = jnp.int32(0)
            for j, v in enumerate(table):
                acc = acc + jnp.where(idx == j, jnp.int32(v), jnp.int32(0))
            return acc

        my_r = lut(RING_INV, my)
        left = lut(RING, m16(my_r - 1))
        right = lut(RING, m16(my_r + 1))

        def erow(c):
            return c * (2 * SQ)

        def orow(c):
            return c * (2 * SQ) + SQ

        barrier = pltpu.get_barrier_semaphore()
        for nbr in (left, right):
            pl.semaphore_signal(barrier, inc=1, device_id=(nbr,),
                                device_id_type=pl.DeviceIdType.MESH)
        pl.semaphore_wait(barrier, 2)

        xg[pl.ds(erow(my), SQ), :] = x_ref[0, :, :].astype(jnp.bfloat16)
        xg[pl.ds(orow(my), SQ), :] = x_ref[1, :, :].astype(jnp.bfloat16)

        rows_i = lax.broadcasted_iota(jnp.int32, (SQ, DH_LOC), 0)
        cols_i = lax.broadcasted_iota(jnp.int32, (SQ, DH_LOC), 1)
        k2 = ((jnp.remainder(cols_i, DH) // 2) * 2).astype(jnp.float32)
        inv = jnp.exp(k2 * (-math.log(10000.0) / DH))
        ang = rows_i.astype(jnp.float32) * inv
        cosv = jnp.cos(ang)
        sinv = jnp.sin(ang)

        ri = lax.broadcasted_iota(jnp.int32, (DH_LOC, DH_LOC), 0)
        ci = lax.broadcasted_iota(jnp.int32, (DH_LOC, DH_LOC), 1)
        even_c = jnp.remainder(ci, 2) == 0
        rot_m = (jnp.where(even_c & (ri == ci + 1), -1.0, 0.0)
                 + jnp.where((~even_c) & (ri == ci - 1), 1.0, 0.0)
                 ).astype(jnp.float32)

        cos2 = jnp.concatenate([cosv, cosv], axis=0)
        sin2 = jnp.concatenate([sinv, sinv], axis=0)
        cos2q = cos2 * 0.125
        sin2q = sin2 * 0.125

        wq = wq_ref[...].astype(jnp.bfloat16)
        wk = wk_ref[...].astype(jnp.bfloat16)
        wv = wv_ref[...].astype(jnp.bfloat16)
        wo = wo_ref[...].astype(jnp.bfloat16)

        def compute_partial(row_e, row_o):
            xb = jnp.concatenate(
                [xg[pl.ds(row_e, SQ), :], xg[pl.ds(row_o, SQ), :]], axis=0)
            q = jnp.dot(xb, wq, preferred_element_type=jnp.float32)
            k = jnp.dot(xb, wk, preferred_element_type=jnp.float32)
            v = jnp.dot(xb, wv, preferred_element_type=jnp.float32)
            q = q * cos2q + jnp.dot(q, rot_m,
                                    preferred_element_type=jnp.float32) * sin2q
            k = k * cos2 + jnp.dot(k, rot_m,
                                   preferred_element_type=jnp.float32) * sin2
            q = q.astype(jnp.bfloat16)
            k = k.astype(jnp.bfloat16)
            v = v.astype(jnp.bfloat16)
            halves = []
            for bb in range(2):
                ctx_heads = []
                for hh in range(H_LOC):
                    qs = q[bb * SQ:(bb + 1) * SQ, hh * DH:(hh + 1) * DH]
                    ks = k[bb * SQ:(bb + 1) * SQ, hh * DH:(hh + 1) * DH]
                    vs = v[bb * SQ:(bb + 1) * SQ, hh * DH:(hh + 1) * DH]
                    s = lax.dot_general(
                        qs, ks, (((1,), (1,)), ((), ())),
                        preferred_element_type=jnp.float32)
                    mx = jnp.max(s, axis=-1, keepdims=True)
                    w = jnp.exp(s - mx)
                    w = (w / jnp.sum(w, axis=-1, keepdims=True)
                         ).astype(jnp.bfloat16)
                    ctx_heads.append(jnp.dot(
                        w, vs,
                        preferred_element_type=jnp.float32
                    ).astype(jnp.bfloat16))
                halves.append(jnp.concatenate(ctx_heads, axis=1))
            ctx = jnp.concatenate(halves, axis=0)
            return jnp.dot(ctx, wo, preferred_element_type=jnp.float32)

        def remote(src, dst, ssem, rsem, dev):
            return pltpu.make_async_remote_copy(
                src_ref=src, dst_ref=dst, send_sem=ssem, recv_sem=rsem,
                device_id=(dev,), device_id_type=pl.DeviceIdType.MESH)

        HD = D // 2

        agr_d = agl_d = rsr_d = rsl_d = None
        for h in range(N_DEV + 1):
            if h < N_DEV:
                re_h = erow(lut(RING, m16(my_r - h)))
                ro_h = orow(lut(RING, m16(my_r + h)))

            if h < N_DEV - 1:
                if agr_d is not None:
                    agr_d.wait_recv()
                new_agr = remote(xg.at[pl.ds(re_h, SQ)],
                                 xg.at[pl.ds(re_h, SQ)],
                                 agr_send.at[h], agr_recv.at[h], right)
                new_agr.start()
                if agl_d is not None:
                    agl_d.wait_recv()
                new_agl = remote(xg.at[pl.ds(ro_h, SQ)],
                                 xg.at[pl.ds(ro_h, SQ)],
                                 agl_send.at[h], agl_recv.at[h], left)
                new_agl.start()
            elif h == N_DEV - 1:
                agr_d.wait_recv()
                agl_d.wait_recv()
                new_agr = new_agl = None

            if h < N_DEV:
                po = compute_partial(re_h, ro_h)
                if h <= 1:
                    part[pl.ds(re_h, SQ), :] = po[0:SQ, :].astype(jnp.bfloat16)
                    part[pl.ds(ro_h, SQ), :] = po[SQ:2 * SQ, :].astype(
                        jnp.bfloat16)

            if h >= 2:
                t = h - 2
                if h < N_DEV:
                    add_e = po[0:SQ, :]
                    add_o = po[SQ:2 * SQ, :]
                else:
                    add_e = part[pl.ds(erow(my), SQ), :].astype(jnp.float32)
                    add_o = part[pl.ds(orow(my), SQ), :].astype(jnp.float32)
                rsr_d.wait_recv()
                rsb_r[t] = (rsb_r[t].astype(jnp.float32)
                            + add_e).astype(jnp.bfloat16)
                rsl_d.wait_recv()
                rsb_l[t] = (rsb_l[t].astype(jnp.float32)
                            + add_o).astype(jnp.bfloat16)
                rsr_d.wait_send()
                rsl_d.wait_send()

            if 1 <= h <= N_DEV - 1:
                t2 = h - 1
                src_r = (part.at[pl.ds(erow(lut(RING, m16(my_r - 1))), SQ)]
                         if t2 == 0 else rsb_r.at[t2 - 1])
                rsr_d = remote(src_r, rsb_r.at[t2],
                               rsr_send.at[t2], rsr_recv.at[t2], right)
                rsr_d.start()
                src_l = (part.at[pl.ds(orow(lut(RING, m16(my_r + 1))), SQ)]
                         if t2 == 0 else rsb_l.at[t2 - 1])
                rsl_d = remote(src_l, rsb_l.at[t2],
                               rsl_send.at[t2], rsl_recv.at[t2], left)
                rsl_d.start()

            if h < N_DEV - 1:
                new_agr.wait_send()
                new_agl.wait_send()
                agr_d, agl_d = new_agr, new_agl

        out_ref[0, :, :] = rsb_r[N_DEV - 2].astype(jnp.float32)
        out_ref[1, :, :] = rsb_l[N_DEV - 2].astype(jnp.float32)

    return pl.pallas_call(
        body,
        out_shape=jax.ShapeDtypeStruct((B_LOC, SQ, D), jnp.float32),
        in_specs=[pl.BlockSpec(memory_space=pltpu.VMEM)] * 5,
        out_specs=pl.BlockSpec(memory_space=pltpu.VMEM),
        scratch_shapes=[
            pltpu.VMEM((ROWS, D), jnp.bfloat16),
            pltpu.VMEM((ROWS, D), jnp.bfloat16),
            pltpu.VMEM((N_DEV - 1, SQ, D), jnp.bfloat16),
            pltpu.VMEM((N_DEV - 1, SQ, D), jnp.bfloat16),
            pltpu.SemaphoreType.DMA((N_DEV - 1,)),
            pltpu.SemaphoreType.DMA((N_DEV - 1,)),
            pltpu.SemaphoreType.DMA((N_DEV - 1,)),
            pltpu.SemaphoreType.DMA((N_DEV - 1,)),
            pltpu.SemaphoreType.DMA((N_DEV - 1,)),
            pltpu.SemaphoreType.DMA((N_DEV - 1,)),
            pltpu.SemaphoreType.DMA((N_DEV - 1,)),
            pltpu.SemaphoreType.DMA((N_DEV - 1,)),
        ],
        compiler_params=pltpu.CompilerParams(
            collective_id=0,
            vmem_limit_bytes=63 * 1024 * 1024,
        ),
    )(x, Wq, Wk, Wv, Wo)
